# baseline (speedup 1.0000x reference)
import functools
import math

import jax
import jax.numpy as jnp
from jax import lax
from jax.experimental import pallas as pl
from jax.experimental.pallas import tpu as pltpu


def _round_up(n, m):
    return ((n + m - 1) // m) * m


def _layer_kernel(xc_ref, xn_ref, w_ref, b_ref, sc_ref, sh_ref, o_ref,
                  xbuf, cbuf, *, BB, cin8, cout, na, P, pout, NU, kc, kcpad,
                  lconv, ntiles):
    xbuf[:, :, 0:P] = xc_ref[...]
    xbuf[:, :, P:P + 128] = xn_ref[...]
    for bb in range(BB):
        for m in range(na):
            cbuf[m * cin8:(m + 1) * cin8, bb * NU:bb * NU + NU] = \
                xbuf[bb, :, m:m + NU]
    if kcpad > kc:
        cbuf[kc:kcpad, :] = jnp.zeros((kcpad - kc, BB * NU), cbuf.dtype)

    y = jnp.dot(w_ref[...], cbuf[...], preferred_element_type=jnp.float32)

    p0 = pl.program_id(1) * pout
    ustar = (lconv - 1) // 8 + 4
    vmax = (lconv - 1) % 8
    ts = ustar - (ntiles - 1) * pout
    bcol = b_ref[0:cout]
    scol = sc_ref[0:cout]
    tcol = sh_ref[0:cout]
    u = p0 + lax.broadcasted_iota(jnp.int32, (cout, NU), 1)
    col_invalid = (u < 4) | (u > ustar)
    col_mixed = u == ustar

    for bb in range(BB):
        yb = y[:, bb * NU:bb * NU + NU]
        zx, zn = yb, yb
        h = 4 * cout
        while h >= cout:
            zx = jnp.maximum(zx[0:h, :], zx[h:2 * h, :])
            zn = jnp.minimum(zn[0:h, :], zn[h:2 * h, :])
            h //= 2
        if 0 <= ts < NU and vmax < 7:
            px = yb[0:cout, ts:ts + 1]
            pn = yb[0:cout, ts:ts + 1]
            for g in range(1, vmax + 1):
                px = jnp.maximum(px, yb[g * cout:(g + 1) * cout, ts:ts + 1])
                pn = jnp.minimum(pn, yb[g * cout:(g + 1) * cout, ts:ts + 1])
            zx = jnp.where(col_mixed, px, zx)
            zn = jnp.where(col_mixed, pn, zn)
        ex = jnp.maximum(zx + bcol, 0.0)
        en = jnp.maximum(zn + bcol, 0.0)
        f = jnp.where(scol >= 0.0, ex, en) * scol + tcol
        f = jnp.where(col_invalid, -1e30, f)
        m1 = jnp.maximum(f[:, 0:NU - 1], f[:, 1:NU])
        m2 = jnp.maximum(m1[:, 0:NU - 3], m1[:, 2:NU - 1])
        m4 = jnp.maximum(m2[:, 0:NU - 7], m2[:, 4:NU - 3])
        o_ref[bb] = m4[:, 0:pout].astype(o_ref.dtype)


def _polyphase(x, off, lt8):
    B, C, L = x.shape
    LT = 8 * lt8
    xp = jnp.pad(x.astype(jnp.bfloat16), ((0, 0), (0, 0), (off, LT - off - L)))
    return xp.reshape(B, C, lt8, 8).transpose(0, 1, 3, 2).reshape(B, 8 * C, lt8)


def _poly_weights(w, kcpad):
    Cout, Cin, K = w.shape
    A = K // 8
    v = jnp.arange(8).reshape(8, 1, 1)
    m = jnp.arange(A + 1).reshape(1, A + 1, 1)
    d = jnp.arange(8).reshape(1, 1, 8)
    b = (d - v) % 8
    c = (d < v).astype(jnp.int32)
    a = m - c
    valid = (a >= 0) & (a < A)
    k = jnp.clip(8 * a + b, 0, K - 1)
    g = w[:, :, k]
    g = jnp.where(valid[None, None], g, 0.0)
    w2 = g.transpose(2, 0, 3, 1, 4).reshape(8 * Cout, (A + 1) * Cin * 8)
    kc = w2.shape[1]
    if kcpad > kc:
        w2 = jnp.pad(w2, ((0, 0), (0, kcpad - kc)))
    return w2.astype(jnp.bfloat16)


def _fused_layer(xpoly, w2, bias, scale, shift, *, BB, cin, cout, K, P, pout,
                 ntiles, lconv, lpool, out_dtype):
    B = xpoly.shape[0]
    cin8 = 8 * cin
    na = K // 8 + 1
    kc = na * cin8
    kcpad = _round_up(kc, 16)
    NU = pout + 8

    assert lpool - (ntiles - 1) * pout > 4
    body = functools.partial(_layer_kernel, BB=BB, cin8=cin8, cout=cout,
                             na=na, P=P, pout=pout, NU=NU, kc=kc,
                             kcpad=kcpad, lconv=lconv, ntiles=ntiles)
    tile8 = jnp.tile
    return pl.pallas_call(
        body,
        out_shape=jax.ShapeDtypeStruct((B, cout, lpool), out_dtype),
        grid=(B // BB, ntiles),
        in_specs=[
            pl.BlockSpec((BB, cin8, P), lambda bi, ti: (bi, 0, ti)),
            pl.BlockSpec((BB, cin8, 128),
                         lambda bi, ti: (bi, 0, (ti + 1) * (P // 128))),
            pl.BlockSpec((8 * cout, kcpad), lambda bi, ti: (0, 0)),
            pl.BlockSpec((8 * cout, 1), lambda bi, ti: (0, 0)),
            pl.BlockSpec((8 * cout, 1), lambda bi, ti: (0, 0)),
            pl.BlockSpec((8 * cout, 1), lambda bi, ti: (0, 0)),
        ],
        out_specs=pl.BlockSpec((BB, cout, pout), lambda bi, ti: (bi, 0, ti)),
        scratch_shapes=[
            pltpu.VMEM((BB, cin8, P + 128), jnp.bfloat16),
            pltpu.VMEM((kcpad, BB * NU), jnp.bfloat16),
        ],
        compiler_params=pltpu.CompilerParams(
            dimension_semantics=("parallel", "arbitrary"),
            vmem_limit_bytes=100 * 1024 * 1024),
    )(xpoly, xpoly, w2,
      tile8(bias, 8).reshape(8 * cout, 1),
      tile8(scale, 8).reshape(8 * cout, 1),
      tile8(shift, 8).reshape(8 * cout, 1))


def _fc_kernel(x_ref, w1_ref, b1_ref, w2_ref, b2_ref, o_ref):
    h = jnp.dot(x_ref[...], w1_ref[...],
                preferred_element_type=jnp.float32) + b1_ref[...]
    h = jnp.maximum(h, 0.0)
    z = jnp.dot(h, w2_ref[...], preferred_element_type=jnp.float32) + b2_ref[...]
    o_ref[...] = 1.0 / (1.0 + jnp.exp(-z))


def _fc_head(x, w1, b1, w2, b2):
    B, F = x.shape
    H = w1.shape[1]
    O = w2.shape[1]
    return pl.pallas_call(
        _fc_kernel,
        out_shape=jax.ShapeDtypeStruct((B, O), jnp.float32),
        grid=(1,),
        in_specs=[
            pl.BlockSpec((B, F), lambda i: (0, 0)),
            pl.BlockSpec((F, H), lambda i: (0, 0)),
            pl.BlockSpec((1, H), lambda i: (0, 0)),
            pl.BlockSpec((H, O), lambda i: (0, 0)),
            pl.BlockSpec((1, O), lambda i: (0, 0)),
        ],
        out_specs=pl.BlockSpec((B, O), lambda i: (0, 0)),
    )(x, w1, b1.reshape(1, H), w2, b2.reshape(1, O))


_LAYERS = (
    dict(K=128, P=1664, BB=16),
    dict(K=64,  P=1152, BB=8),
    dict(K=256, P=384,  BB=16),
)


def kernel(x, w1, b1, w2, b2, w3, b3, g1, beta1, mean1, var1,
           g2, beta2, mean2, var2, g3, beta3, mean3, var3,
           fw1, fb1, fw2, fb2):
    eps = 1e-5
    ws = (w1, w2, w3)
    bs = (b1, b2, b3)
    gs = (g1, g2, g3)
    betas = (beta1, beta2, beta3)
    means = (mean1, mean2, mean3)
    vars_ = (var1, var2, var3)

    h = x[:, None, :].astype(jnp.float32)
    for i, cfg in enumerate(_LAYERS):
        K, P, BB = cfg["K"], cfg["P"], cfg["BB"]
        BB = math.gcd(BB, h.shape[0])
        w = ws[i]
        cout, cin, _ = w.shape
        Lin = h.shape[2]
        lconv = Lin + 1
        lpool = lconv // 8 + 1
        ntiles = pl.cdiv(lpool, P)
        pout = lpool if ntiles == 1 else P
        lt8 = ntiles * P + 128
        off = 32 + K // 2
        scale = gs[i] / jnp.sqrt(vars_[i] + eps)
        shift = betas[i] - means[i] * scale
        xp = _polyphase(h, off, lt8)
        w2p = _poly_weights(w, _round_up((K // 8 + 1) * cin * 8, 16))
        out_dtype = jnp.float32 if i == 2 else jnp.bfloat16
        h = _fused_layer(xp, w2p, bs[i], scale, shift, BB=BB, cin=cin,
                         cout=cout, K=K, P=P, pout=pout, ntiles=ntiles,
                         lconv=lconv, lpool=lpool, out_dtype=out_dtype)

    B = h.shape[0]
    flat = h.reshape(B, -1)
    return _fc_head(flat.astype(jnp.bfloat16), fw1.astype(jnp.bfloat16),
                    fb1, fw2, fb2)


# speedup vs baseline: 1.0165x; 1.0165x over previous
import functools
import math

import jax
import jax.numpy as jnp
from jax import lax
from jax.experimental import pallas as pl
from jax.experimental.pallas import tpu as pltpu


def _round_up(n, m):
    return ((n + m - 1) // m) * m


def _layer_kernel(xc_ref, xn_ref, w_ref, b_ref, sc_ref, sh_ref, o_ref,
                  xbuf, cbuf, *, BB, cin8, cout, na, P, pout, NU, kc, kcpad,
                  lconv, ntiles):
    xbuf[:, :, 0:P] = xc_ref[...]
    xbuf[:, :, P:P + 128] = xn_ref[...]
    for bb in range(BB):
        for m in range(na):
            cbuf[m * cin8:(m + 1) * cin8, bb * NU:bb * NU + NU] = \
                xbuf[bb, :, m:m + NU]
    if kcpad > kc:
        cbuf[kc:kcpad, :] = jnp.zeros((kcpad - kc, BB * NU), cbuf.dtype)

    y = jnp.dot(w_ref[...], cbuf[...], preferred_element_type=jnp.float32)

    p0 = pl.program_id(1) * pout
    ustar = (lconv - 1) // 8 + 4
    vmax = (lconv - 1) % 8
    ts = ustar - (ntiles - 1) * pout
    bcol = b_ref[0:cout]
    scol = sc_ref[0:cout]
    tcol = sh_ref[0:cout]
    u = p0 + lax.broadcasted_iota(jnp.int32, (cout, NU), 1)
    col_invalid = (u < 4) | (u > ustar)
    col_mixed = u == ustar

    for bb in range(BB):
        yb = y[:, bb * NU:bb * NU + NU]
        zx, zn = yb, yb
        h = 4 * cout
        while h >= cout:
            zx = jnp.maximum(zx[0:h, :], zx[h:2 * h, :])
            zn = jnp.minimum(zn[0:h, :], zn[h:2 * h, :])
            h //= 2
        if 0 <= ts < NU and vmax < 7:
            px = yb[0:cout, ts:ts + 1]
            pn = yb[0:cout, ts:ts + 1]
            for g in range(1, vmax + 1):
                px = jnp.maximum(px, yb[g * cout:(g + 1) * cout, ts:ts + 1])
                pn = jnp.minimum(pn, yb[g * cout:(g + 1) * cout, ts:ts + 1])
            zx = jnp.where(col_mixed, px, zx)
            zn = jnp.where(col_mixed, pn, zn)
        ex = jnp.maximum(zx + bcol, 0.0)
        en = jnp.maximum(zn + bcol, 0.0)
        f = jnp.where(scol >= 0.0, ex, en) * scol + tcol
        f = jnp.where(col_invalid, -1e30, f)
        m1 = jnp.maximum(f[:, 0:NU - 1], f[:, 1:NU])
        m2 = jnp.maximum(m1[:, 0:NU - 3], m1[:, 2:NU - 1])
        m4 = jnp.maximum(m2[:, 0:NU - 7], m2[:, 4:NU - 3])
        o_ref[bb] = m4[:, 0:pout].astype(o_ref.dtype)


def _polyphase(x, off, lt8):
    B, C, L = x.shape
    LT = 8 * lt8
    xp = jnp.pad(x.astype(jnp.bfloat16), ((0, 0), (0, 0), (off, LT - off - L)))
    return xp.reshape(B, C, lt8, 8).transpose(0, 1, 3, 2).reshape(B, 8 * C, lt8)


def _poly_weights(w, kcpad):
    Cout, Cin, K = w.shape
    A = K // 8
    v = jnp.arange(8).reshape(8, 1, 1)
    m = jnp.arange(A + 1).reshape(1, A + 1, 1)
    d = jnp.arange(8).reshape(1, 1, 8)
    b = (d - v) % 8
    c = (d < v).astype(jnp.int32)
    a = m - c
    valid = (a >= 0) & (a < A)
    k = jnp.clip(8 * a + b, 0, K - 1)
    g = w[:, :, k]
    g = jnp.where(valid[None, None], g, 0.0)
    w2 = g.transpose(2, 0, 3, 1, 4).reshape(8 * Cout, (A + 1) * Cin * 8)
    kc = w2.shape[1]
    if kcpad > kc:
        w2 = jnp.pad(w2, ((0, 0), (0, kcpad - kc)))
    return w2.astype(jnp.bfloat16)


def _fused_layer(xpoly, w2, bias, scale, shift, *, BB, cin, cout, K, P, pout,
                 ntiles, lconv, lpool, out_dtype):
    B = xpoly.shape[0]
    cin8 = 8 * cin
    na = K // 8 + 1
    kc = na * cin8
    kcpad = _round_up(kc, 16)
    NU = pout + 8

    assert lpool - (ntiles - 1) * pout > 4
    body = functools.partial(_layer_kernel, BB=BB, cin8=cin8, cout=cout,
                             na=na, P=P, pout=pout, NU=NU, kc=kc,
                             kcpad=kcpad, lconv=lconv, ntiles=ntiles)
    tile8 = jnp.tile
    return pl.pallas_call(
        body,
        out_shape=jax.ShapeDtypeStruct((B, cout, lpool), out_dtype),
        grid=(B // BB, ntiles),
        in_specs=[
            pl.BlockSpec((BB, cin8, P), lambda bi, ti: (bi, 0, ti)),
            pl.BlockSpec((BB, cin8, 128),
                         lambda bi, ti: (bi, 0, (ti + 1) * (P // 128))),
            pl.BlockSpec((8 * cout, kcpad), lambda bi, ti: (0, 0)),
            pl.BlockSpec((8 * cout, 1), lambda bi, ti: (0, 0)),
            pl.BlockSpec((8 * cout, 1), lambda bi, ti: (0, 0)),
            pl.BlockSpec((8 * cout, 1), lambda bi, ti: (0, 0)),
        ],
        out_specs=pl.BlockSpec((BB, cout, pout), lambda bi, ti: (bi, 0, ti)),
        scratch_shapes=[
            pltpu.VMEM((BB, cin8, P + 128), jnp.bfloat16),
            pltpu.VMEM((kcpad, BB * NU), jnp.bfloat16),
        ],
        compiler_params=pltpu.CompilerParams(
            dimension_semantics=("parallel", "arbitrary"),
            vmem_limit_bytes=100 * 1024 * 1024),
    )(xpoly, xpoly, w2,
      tile8(bias, 8).reshape(8 * cout, 1),
      tile8(scale, 8).reshape(8 * cout, 1),
      tile8(shift, 8).reshape(8 * cout, 1))


def _fc_kernel(x_ref, w1_ref, b1_ref, w2_ref, b2_ref, o_ref):
    h = jnp.dot(x_ref[...], w1_ref[...],
                preferred_element_type=jnp.float32) + b1_ref[...]
    h = jnp.maximum(h, 0.0)
    z = jnp.dot(h, w2_ref[...], preferred_element_type=jnp.float32) + b2_ref[...]
    o_ref[...] = 1.0 / (1.0 + jnp.exp(-z))


def _fc_head(x, w1, b1, w2, b2):
    B, F = x.shape
    H = w1.shape[1]
    O = w2.shape[1]
    return pl.pallas_call(
        _fc_kernel,
        out_shape=jax.ShapeDtypeStruct((B, O), jnp.float32),
        grid=(1,),
        in_specs=[
            pl.BlockSpec((B, F), lambda i: (0, 0)),
            pl.BlockSpec((F, H), lambda i: (0, 0)),
            pl.BlockSpec((1, H), lambda i: (0, 0)),
            pl.BlockSpec((H, O), lambda i: (0, 0)),
            pl.BlockSpec((1, O), lambda i: (0, 0)),
        ],
        out_specs=pl.BlockSpec((B, O), lambda i: (0, 0)),
    )(x, w1, b1.reshape(1, H), w2, b2.reshape(1, O))


_LAYERS = (
    dict(K=128, P=1664, BB=16),
    dict(K=64,  P=1152, BB=4),
    dict(K=256, P=384,  BB=16),
)


def kernel(x, w1, b1, w2, b2, w3, b3, g1, beta1, mean1, var1,
           g2, beta2, mean2, var2, g3, beta3, mean3, var3,
           fw1, fb1, fw2, fb2):
    eps = 1e-5
    ws = (w1, w2, w3)
    bs = (b1, b2, b3)
    gs = (g1, g2, g3)
    betas = (beta1, beta2, beta3)
    means = (mean1, mean2, mean3)
    vars_ = (var1, var2, var3)

    h = x[:, None, :].astype(jnp.float32)
    for i, cfg in enumerate(_LAYERS):
        K, P, BB = cfg["K"], cfg["P"], cfg["BB"]
        BB = math.gcd(BB, h.shape[0])
        w = ws[i]
        cout, cin, _ = w.shape
        Lin = h.shape[2]
        lconv = Lin + 1
        lpool = lconv // 8 + 1
        ntiles = pl.cdiv(lpool, P)
        pout = lpool if ntiles == 1 else P
        lt8 = ntiles * P + 128
        off = 32 + K // 2
        scale = gs[i] / jnp.sqrt(vars_[i] + eps)
        shift = betas[i] - means[i] * scale
        xp = _polyphase(h, off, lt8)
        w2p = _poly_weights(w, _round_up((K // 8 + 1) * cin * 8, 16))
        out_dtype = jnp.float32 if i == 2 else jnp.bfloat16
        h = _fused_layer(xp, w2p, bs[i], scale, shift, BB=BB, cin=cin,
                         cout=cout, K=K, P=P, pout=pout, ntiles=ntiles,
                         lconv=lconv, lpool=lpool, out_dtype=out_dtype)

    B = h.shape[0]
    flat = h.reshape(B, -1)
    return _fc_head(flat.astype(jnp.bfloat16), fw1.astype(jnp.bfloat16),
                    fb1, fw2, fb2)
